# Initial kernel scaffold; baseline (speedup 1.0000x reference)
import jax
import jax.numpy as jnp
from jax.experimental import pallas as pl


def kernel(atomic_numbers):
    raise NotImplementedError("write your pallas kernel here")


# speedup vs baseline: 10.3616x; 10.3616x over previous
import jax
import jax.numpy as jnp
from jax.experimental import pallas as pl

N_ROWS = 100000
N_COLS = 100
ROWS_PER_BLOCK = 1000


def _onehot_body(idx_ref, out_ref):
    idx = idx_ref[0, 0, :]
    cols = jax.lax.broadcasted_iota(jnp.int32, (ROWS_PER_BLOCK, N_COLS), 1)
    out_ref[...] = (cols == idx[:, None]).astype(jnp.float32)


def kernel(atomic_numbers):
    n_blocks = N_ROWS // ROWS_PER_BLOCK
    idx3 = atomic_numbers.reshape(n_blocks, 1, ROWS_PER_BLOCK)
    return pl.pallas_call(
        _onehot_body,
        grid=(n_blocks,),
        in_specs=[pl.BlockSpec((1, 1, ROWS_PER_BLOCK), lambda i: (i, 0, 0))],
        out_specs=pl.BlockSpec((ROWS_PER_BLOCK, N_COLS), lambda i: (i, 0)),
        out_shape=jax.ShapeDtypeStruct((N_ROWS, N_COLS), jnp.float32),
    )(idx3)
